# baseline (speedup 1.0000x reference)
import functools

import jax
import jax.numpy as jnp
from jax import lax
from jax.experimental import pallas as pl
from jax.experimental.pallas import tpu as pltpu
from jax.experimental.pallas import tpu_sc as plsc

_N = 10000
_E = 320000
_D = 128
_H = 128
_CW = 16
_NC = 2
_NS = 16
_NW = _NC * _NS
_EPW = _E // _NW
_C = 125
_K = _EPW // _C
_NPT = _N // _NS
_SS = 16
_NSS = _K // _SS
_EPS = 1e-5


def _colstats(y):
    ones = jnp.full((8, _N), 1.0 / _N, dtype=jnp.float32)
    m = jnp.dot(ones, y, preferred_element_type=jnp.float32,
                precision=jax.lax.Precision.HIGHEST)[0:1]
    q = jnp.dot(ones, y * y, preferred_element_type=jnp.float32,
                precision=jax.lax.Precision.HIGHEST)[0:1]
    return m, q - m * m


def _tc1_body(x_ref, w1_ref, b1_ref, g1_ref, be1_ref, feat_ref):
    y = jnp.dot(x_ref[...], w1_ref[...], preferred_element_type=jnp.float32)
    y = y + b1_ref[...]
    m, v = _colstats(y)
    feat = (y - m) * jax.lax.rsqrt(v + _EPS) * g1_ref[...] + be1_ref[...]
    feat_ref[...] = jnp.maximum(feat, 0.0)


def _tc2_body(part_ref, cnt_ref, feat_ref, wl_ref, bl_ref, wr_ref, g2_ref,
              be2_ref, out_ref):
    agg = part_ref[0] + part_ref[1]
    cnt = cnt_ref[0, :, :1] + cnt_ref[1, :, :1]
    mean_agg = agg / jnp.maximum(cnt, 1.0)
    sage = jnp.dot(mean_agg, wl_ref[...], preferred_element_type=jnp.float32)
    sage = sage + bl_ref[...]
    sage = sage + jnp.dot(feat_ref[...], wr_ref[...],
                          preferred_element_type=jnp.float32)
    m, v = _colstats(sage)
    out_ref[...] = ((sage - m) * jax.lax.rsqrt(v + _EPS) * g2_ref[...]
                    + be2_ref[...])


def _sc_seg_body(edge_hbm, feat_hbm, out_hbm, cnt_hbm,
                 src_v, dst_v, rows_v, rows_w, ones_v, acc_sh, cnt_sh,
                 sema, semb):
    c = lax.axis_index("c")
    s = lax.axis_index("s")
    wid = s * _NC + c
    src_hbm = edge_hbm.at[0]
    dst_hbm = edge_hbm.at[1]

    def _zero_row(r, _):
        for j in range(_H // 16):
            rows_v[r, pl.ds(j * 16, 16)] = jnp.zeros((16,), jnp.float32)
        ones_v[r, pl.ds(0, 16)] = jnp.zeros((16,), jnp.float32)
        return 0
    lax.fori_loop(0, _C, _zero_row, 0)
    for q in range(_NPT // _C):
        pltpu.sync_copy(rows_v, acc_sh.at[pl.ds(s * _NPT + q * _C, _C)])
        pltpu.sync_copy(ones_v, cnt_sh.at[pl.ds(s * _NPT + q * _C, _C)])

    def _ones_row(r, _):
        ones_v[r, pl.ds(0, 16)] = jnp.ones((16,), jnp.float32)
        return 0
    lax.fori_loop(0, _C, _ones_row, 0)
    plsc.subcore_barrier()

    def _super(ss, _):
        pltpu.sync_copy(src_hbm.at[wid].at[pl.ds(ss * _SS, _SS)], src_v)
        pltpu.sync_copy(dst_hbm.at[wid].at[pl.ds(ss * _SS, _SS)], dst_v)
        pltpu.async_copy(feat_hbm.at[src_v.at[0]], rows_v, sema)

        def _chunk2(i, _):
            t0 = 2 * i
            t1 = t0 + 1
            pltpu.make_async_copy(feat_hbm.at[src_v.at[t0]], rows_v,
                                  sema).wait()
            pltpu.async_copy(feat_hbm.at[src_v.at[t1]], rows_w, semb)
            pltpu.sync_copy(rows_v, acc_sh.at[dst_v.at[t0]], add=True)
            pltpu.sync_copy(ones_v, cnt_sh.at[dst_v.at[t0]], add=True)

            @pl.when(t0 + 2 < _SS)
            def _():
                pltpu.async_copy(feat_hbm.at[src_v.at[t0 + 2]], rows_v, sema)

            pltpu.make_async_copy(feat_hbm.at[src_v.at[t1]], rows_w,
                                  semb).wait()
            pltpu.sync_copy(rows_w, acc_sh.at[dst_v.at[t1]], add=True)
            pltpu.sync_copy(ones_v, cnt_sh.at[dst_v.at[t1]], add=True)
            return 0
        lax.fori_loop(0, _SS // 2, _chunk2, 0)
        return 0
    lax.fori_loop(0, _NSS, _super, 0)

    plsc.subcore_barrier()
    pltpu.sync_copy(acc_sh.at[pl.ds(s * _NPT, _NPT)],
                    out_hbm.at[c].at[pl.ds(s * _NPT, _NPT)])
    pltpu.sync_copy(cnt_sh.at[pl.ds(s * _NPT, _NPT)],
                    cnt_hbm.at[c].at[pl.ds(s * _NPT, _NPT)])


@functools.cache
def _make_sc_seg():
    return functools.partial(
        pl.kernel,
        out_type=(jax.ShapeDtypeStruct((_NC, _N, _H), jnp.float32),
                  jax.ShapeDtypeStruct((_NC, _N, _CW), jnp.float32)),
        mesh=plsc.VectorSubcoreMesh(core_axis_name="c", subcore_axis_name="s",
                                    num_cores=_NC, num_subcores=_NS),
        scratch_types=[
            pltpu.VMEM((_SS, _C), jnp.int32),
            pltpu.VMEM((_SS, _C), jnp.int32),
            pltpu.VMEM((_C, _H), jnp.float32),
            pltpu.VMEM((_C, _H), jnp.float32),
            pltpu.VMEM((_C, _CW), jnp.float32),
            pltpu.VMEM_SHARED((_N, _H), jnp.float32),
            pltpu.VMEM_SHARED((_N, _CW), jnp.float32),
            pltpu.SemaphoreType.DMA,
            pltpu.SemaphoreType.DMA,
        ],
        compiler_params=pltpu.CompilerParams(use_tc_tiling_on_sc=False),
    )(_sc_seg_body)


def kernel(x, edge_index, W1, b1, g1, be1, Wl, bl, Wr, g2, be2):
    f32 = jnp.float32
    feat = pl.pallas_call(
        _tc1_body,
        out_shape=jax.ShapeDtypeStruct((_N, _H), f32),
    )(x, W1, b1.reshape(1, _H), g1.reshape(1, _H), be1.reshape(1, _H))

    edges = edge_index.reshape(2, _NW, _K, _C)
    parts, cnts = _make_sc_seg()(edges, feat)

    out_feat = pl.pallas_call(
        _tc2_body,
        out_shape=jax.ShapeDtypeStruct((_N, _H), f32),
    )(parts, cnts, feat, Wl, bl.reshape(1, _H), Wr, g2.reshape(1, _H),
      be2.reshape(1, _H))
    return (feat, out_feat)


# speedup vs baseline: 1.0438x; 1.0438x over previous
import functools

import jax
import jax.numpy as jnp
from jax import lax
from jax.experimental import pallas as pl
from jax.experimental.pallas import tpu as pltpu
from jax.experimental.pallas import tpu_sc as plsc

_N = 10000
_E = 320000
_D = 128
_H = 128
_CW = 16
_NC = 2
_NS = 16
_NW = _NC * _NS
_EPW = _E // _NW
_C = 125
_K = _EPW // _C
_NPT = _N // _NS
_SS = 16
_NSS = _K // _SS
_EPS = 1e-5


def _colstats(y):
    m = jnp.mean(y, axis=0, keepdims=True)
    d = y - m
    v = jnp.mean(d * d, axis=0, keepdims=True)
    return m, v


def _tc1_body(x_ref, w1_ref, b1_ref, g1_ref, be1_ref, feat_ref):
    y = jnp.dot(x_ref[...], w1_ref[...], preferred_element_type=jnp.float32)
    y = y + b1_ref[...]
    m, v = _colstats(y)
    feat = (y - m) * jax.lax.rsqrt(v + _EPS) * g1_ref[...] + be1_ref[...]
    feat_ref[...] = jnp.maximum(feat, 0.0)


def _tc2_body(part_ref, cnt_ref, feat_ref, wl_ref, bl_ref, wr_ref, g2_ref,
              be2_ref, out_ref):
    agg = part_ref[0] + part_ref[1]
    cnt = cnt_ref[0, :, :1] + cnt_ref[1, :, :1]
    mean_agg = agg / jnp.maximum(cnt, 1.0)
    sage = jnp.dot(mean_agg, wl_ref[...], preferred_element_type=jnp.float32)
    sage = sage + bl_ref[...]
    sage = sage + jnp.dot(feat_ref[...], wr_ref[...],
                          preferred_element_type=jnp.float32)
    m, v = _colstats(sage)
    out_ref[...] = ((sage - m) * jax.lax.rsqrt(v + _EPS) * g2_ref[...]
                    + be2_ref[...])


def _sc_seg_body(edge_hbm, feat_hbm, out_hbm, cnt_hbm,
                 src_v, dst_v, rows_v, rows_w, ones_v, acc_sh, cnt_sh,
                 sema, semb):
    c = lax.axis_index("c")
    s = lax.axis_index("s")
    wid = s * _NC + c
    src_hbm = edge_hbm.at[0]
    dst_hbm = edge_hbm.at[1]

    def _zero_row(r, _):
        for j in range(_H // 16):
            rows_v[r, pl.ds(j * 16, 16)] = jnp.zeros((16,), jnp.float32)
        ones_v[r, pl.ds(0, 16)] = jnp.zeros((16,), jnp.float32)
        return 0
    lax.fori_loop(0, _C, _zero_row, 0)
    for q in range(_NPT // _C):
        pltpu.sync_copy(rows_v, acc_sh.at[pl.ds(s * _NPT + q * _C, _C)])
        pltpu.sync_copy(ones_v, cnt_sh.at[pl.ds(s * _NPT + q * _C, _C)])

    def _ones_row(r, _):
        ones_v[r, pl.ds(0, 16)] = jnp.ones((16,), jnp.float32)
        return 0
    lax.fori_loop(0, _C, _ones_row, 0)
    plsc.subcore_barrier()

    def _super(ss, _):
        pltpu.sync_copy(src_hbm.at[wid].at[pl.ds(ss * _SS, _SS)], src_v)
        pltpu.sync_copy(dst_hbm.at[wid].at[pl.ds(ss * _SS, _SS)], dst_v)
        pltpu.async_copy(feat_hbm.at[src_v.at[0]], rows_v, sema)

        def _chunk2(i, _):
            t0 = 2 * i
            t1 = t0 + 1
            pltpu.make_async_copy(feat_hbm.at[src_v.at[t0]], rows_v,
                                  sema).wait()
            pltpu.async_copy(feat_hbm.at[src_v.at[t1]], rows_w, semb)
            pltpu.sync_copy(rows_v, acc_sh.at[dst_v.at[t0]], add=True)
            pltpu.sync_copy(ones_v, cnt_sh.at[dst_v.at[t0]], add=True)

            @pl.when(t0 + 2 < _SS)
            def _():
                pltpu.async_copy(feat_hbm.at[src_v.at[t0 + 2]], rows_v, sema)

            pltpu.make_async_copy(feat_hbm.at[src_v.at[t1]], rows_w,
                                  semb).wait()
            pltpu.sync_copy(rows_w, acc_sh.at[dst_v.at[t1]], add=True)
            pltpu.sync_copy(ones_v, cnt_sh.at[dst_v.at[t1]], add=True)
            return 0
        lax.fori_loop(0, _SS // 2, _chunk2, 0)
        return 0
    lax.fori_loop(0, _NSS, _super, 0)

    plsc.subcore_barrier()
    pltpu.sync_copy(acc_sh.at[pl.ds(s * _NPT, _NPT)],
                    out_hbm.at[c].at[pl.ds(s * _NPT, _NPT)])
    pltpu.sync_copy(cnt_sh.at[pl.ds(s * _NPT, _NPT)],
                    cnt_hbm.at[c].at[pl.ds(s * _NPT, _NPT)])


@functools.cache
def _make_sc_seg():
    return functools.partial(
        pl.kernel,
        out_type=(jax.ShapeDtypeStruct((_NC, _N, _H), jnp.float32),
                  jax.ShapeDtypeStruct((_NC, _N, _CW), jnp.float32)),
        mesh=plsc.VectorSubcoreMesh(core_axis_name="c", subcore_axis_name="s",
                                    num_cores=_NC, num_subcores=_NS),
        scratch_types=[
            pltpu.VMEM((_SS, _C), jnp.int32),
            pltpu.VMEM((_SS, _C), jnp.int32),
            pltpu.VMEM((_C, _H), jnp.float32),
            pltpu.VMEM((_C, _H), jnp.float32),
            pltpu.VMEM((_C, _CW), jnp.float32),
            pltpu.VMEM_SHARED((_N, _H), jnp.float32),
            pltpu.VMEM_SHARED((_N, _CW), jnp.float32),
            pltpu.SemaphoreType.DMA,
            pltpu.SemaphoreType.DMA,
        ],
        compiler_params=pltpu.CompilerParams(use_tc_tiling_on_sc=False),
    )(_sc_seg_body)


def kernel(x, edge_index, W1, b1, g1, be1, Wl, bl, Wr, g2, be2):
    f32 = jnp.float32
    feat = pl.pallas_call(
        _tc1_body,
        out_shape=jax.ShapeDtypeStruct((_N, _H), f32),
    )(x, W1, b1.reshape(1, _H), g1.reshape(1, _H), be1.reshape(1, _H))

    edges = edge_index.reshape(2, _NW, _K, _C)
    parts, cnts = _make_sc_seg()(edges, feat)

    out_feat = pl.pallas_call(
        _tc2_body,
        out_shape=jax.ShapeDtypeStruct((_N, _H), f32),
    )(parts, cnts, feat, Wl, bl.reshape(1, _H), Wr, g2.reshape(1, _H),
      be2.reshape(1, _H))
    return (feat, out_feat)
